# Initial kernel scaffold; baseline (speedup 1.0000x reference)
import functools

import jax
import jax.numpy as jnp
from jax import lax
from jax.experimental import pallas as pl
from jax.experimental.pallas import tpu as pltpu
from jax.experimental.pallas import tpu_sc as plsc

N, E, F, H, EA, V = 10000, 320000, 128, 128, 16, 3

NC, NS = 2, 16
NW = NC * NS
E_LOCAL = E // NW
CH = 80
NCHUNK = E_LOCAL // CH
ROWS_PER_TILE = N // NS

_f32 = jnp.float32


def _silu(z):
    return z * jax.nn.sigmoid(z)


def _t0_body(h_ref, w1a_ref, w1b_ref, a_ref, b_ref):
    hb = h_ref[...]
    a_ref[...] = jnp.dot(hb, w1a_ref[...], preferred_element_type=_f32)
    b_ref[...] = jnp.dot(hb, w1b_ref[...], preferred_element_type=_f32)


def _t0(h, w1a, w1b, bn=2000):
    grid = (N // bn,)
    return pl.pallas_call(
        _t0_body,
        grid=grid,
        in_specs=[
            pl.BlockSpec((bn, F), lambda i: (i, 0)),
            pl.BlockSpec((F, H), lambda i: (0, 0)),
            pl.BlockSpec((F, H), lambda i: (0, 0)),
        ],
        out_specs=[
            pl.BlockSpec((bn, H), lambda i: (i, 0)),
            pl.BlockSpec((bn, H), lambda i: (i, 0)),
        ],
        out_shape=[
            jax.ShapeDtypeStruct((N, H), _f32),
            jax.ShapeDtypeStruct((N, H), _f32),
        ],
    )(h, w1a, w1b)


def _sc_gather(a, b, x, src, dst):
    mesh = plsc.VectorSubcoreMesh(core_axis_name="c", subcore_axis_name="s")

    @functools.partial(
        pl.kernel,
        mesh=mesh,
        out_type=(
            jax.ShapeDtypeStruct((E, H), _f32),
            jax.ShapeDtypeStruct((E, H), _f32),
            jax.ShapeDtypeStruct((E, V), _f32),
            jax.ShapeDtypeStruct((E, V), _f32),
        ),
        scratch_types=[
            pltpu.VMEM((CH,), jnp.int32),
            pltpu.VMEM((CH,), jnp.int32),
            pltpu.VMEM((CH, H), _f32),
            pltpu.VMEM((CH, H), _f32),
            pltpu.VMEM((CH, V), _f32),
            pltpu.VMEM((CH, V), _f32),
            pltpu.SemaphoreType.DMA,
            pltpu.SemaphoreType.DMA,
        ],
    )
    def kern(a_hbm, b_hbm, x_hbm, src_hbm, dst_hbm,
             gs_out, gd_out, xs_out, xd_out,
             src_v, dst_v, buf_a, buf_b, buf_xs, buf_xd, sem_g, sem_w):
        wid = lax.axis_index("s") * NC + lax.axis_index("c")
        base = wid * E_LOCAL

        def chunk(i, carry):
            off = base + i * CH
            pltpu.sync_copy(src_hbm.at[pl.ds(off, CH)], src_v)
            pltpu.sync_copy(dst_hbm.at[pl.ds(off, CH)], dst_v)
            g1 = pltpu.async_copy(a_hbm.at[src_v], buf_a, sem_g)
            g2 = pltpu.async_copy(b_hbm.at[dst_v], buf_b, sem_g)
            g3 = pltpu.async_copy(x_hbm.at[src_v], buf_xs, sem_g)
            g4 = pltpu.async_copy(x_hbm.at[dst_v], buf_xd, sem_g)
            g1.wait()
            g2.wait()
            g3.wait()
            g4.wait()
            w1 = pltpu.async_copy(buf_a, gs_out.at[pl.ds(off, CH)], sem_w)
            w2 = pltpu.async_copy(buf_b, gd_out.at[pl.ds(off, CH)], sem_w)
            w3 = pltpu.async_copy(buf_xs, xs_out.at[pl.ds(off, CH)], sem_w)
            w4 = pltpu.async_copy(buf_xd, xd_out.at[pl.ds(off, CH)], sem_w)
            w1.wait()
            w2.wait()
            w3.wait()
            w4.wait()
            return carry

        lax.fori_loop(0, NCHUNK, chunk, 0)

    return kern(a, b, x, src, dst)


def _t1_body(gs_ref, gd_ref, ea_ref, xs_ref, xd_ref,
             w1d_ref, w1c_ref, b1_ref, w2_ref, b2_ref,
             cfw1_ref, cfb1_ref, cfw2_ref, cfb2_ref, cs_ref,
             m_ref, cu_ref):
    xdiff = xd_ref[...] - xs_ref[...]
    xsq = jnp.sum(xdiff * xdiff, axis=1, keepdims=True)
    z1 = (gs_ref[...] + gd_ref[...]
          + jnp.dot(ea_ref[...], w1d_ref[...], preferred_element_type=_f32)
          + xsq * w1c_ref[...]
          + b1_ref[...])
    m1 = _silu(z1)
    t2 = jnp.dot(m1, w2_ref[...], preferred_element_type=_f32) + b2_ref[...]
    m_ij = _silu(t2)
    g1 = jnp.dot(m_ij, cfw1_ref[...], preferred_element_type=_f32) + cfb1_ref[...]
    gate = _silu(g1) * cfw2_ref[0, 0] + cfb2_ref[0, 0]
    m_ref[...] = m_ij
    cu_ref[...] = cs_ref[0, 0] * gate * xdiff


def _t1(gs, gd, ea, xs, xd, w1d, w1c, b1, w2, b2, cfw1, cfb1, cfw2, cfb2, cs,
        be=2000):
    grid = (E // be,)
    full = lambda shape: pl.BlockSpec(shape, lambda i: tuple(0 for _ in shape))
    return pl.pallas_call(
        _t1_body,
        grid=grid,
        in_specs=[
            pl.BlockSpec((be, H), lambda i: (i, 0)),
            pl.BlockSpec((be, H), lambda i: (i, 0)),
            pl.BlockSpec((be, EA), lambda i: (i, 0)),
            pl.BlockSpec((be, V), lambda i: (i, 0)),
            pl.BlockSpec((be, V), lambda i: (i, 0)),
            full((EA, H)),
            full((1, H)),
            full((1, H)),
            full((H, F)),
            full((1, F)),
            full((F, 1)),
            full((1, 1)),
            full((1, 1)),
            full((1, 1)),
            full((1, 1)),
        ],
        out_specs=[
            pl.BlockSpec((be, F), lambda i: (i, 0)),
            pl.BlockSpec((be, V), lambda i: (i, 0)),
        ],
        out_shape=[
            jax.ShapeDtypeStruct((E, F), _f32),
            jax.ShapeDtypeStruct((E, V), _f32),
        ],
    )(gs, gd, ea, xs, xd, w1d, w1c, b1, w2, b2, cfw1, cfb1, cfw2, cfb2, cs)


def _sc_scatter(m_ij, cu, dst, zeros_h, zeros_v):
    mesh = plsc.VectorSubcoreMesh(core_axis_name="c", subcore_axis_name="s")

    @functools.partial(
        pl.kernel,
        mesh=mesh,
        out_type=(
            jax.ShapeDtypeStruct((NC, N, H), _f32),
            jax.ShapeDtypeStruct((NC, N, V), _f32),
        ),
        scratch_types=[
            pltpu.VMEM((CH,), jnp.int32),
            pltpu.VMEM((CH, H), _f32),
            pltpu.VMEM((CH, V), _f32),
            pltpu.VMEM_SHARED((N, H), _f32),
            pltpu.VMEM_SHARED((N, V), _f32),
        ],
    )
    def kern(m_hbm, cu_hbm, dst_hbm, zh_hbm, zv_hbm,
             mi_out, vu_out,
             idx_v, buf_m, buf_c, acc_mi, acc_vu):
        cid = lax.axis_index("c")
        sid = lax.axis_index("s")
        wid = sid * NC + cid
        r0 = sid * ROWS_PER_TILE
        pltpu.sync_copy(zh_hbm.at[pl.ds(r0, ROWS_PER_TILE)],
                        acc_mi.at[pl.ds(r0, ROWS_PER_TILE)])
        pltpu.sync_copy(zv_hbm.at[pl.ds(r0, ROWS_PER_TILE)],
                        acc_vu.at[pl.ds(r0, ROWS_PER_TILE)])
        plsc.subcore_barrier()

        base = wid * E_LOCAL

        def chunk(i, carry):
            off = base + i * CH
            pltpu.sync_copy(dst_hbm.at[pl.ds(off, CH)], idx_v)
            pltpu.sync_copy(m_hbm.at[pl.ds(off, CH)], buf_m)
            pltpu.sync_copy(cu_hbm.at[pl.ds(off, CH)], buf_c)
            pltpu.sync_copy(buf_m, acc_mi.at[idx_v], add=True)
            pltpu.sync_copy(buf_c, acc_vu.at[idx_v], add=True)
            return carry

        lax.fori_loop(0, NCHUNK, chunk, 0)
        plsc.subcore_barrier()
        pltpu.sync_copy(acc_mi.at[pl.ds(r0, ROWS_PER_TILE)],
                        mi_out.at[cid, pl.ds(r0, ROWS_PER_TILE)])
        pltpu.sync_copy(acc_vu.at[pl.ds(r0, ROWS_PER_TILE)],
                        vu_out.at[cid, pl.ds(r0, ROWS_PER_TILE)])

    return kern(m_ij, cu, dst, zeros_h, zeros_v)


def _t2_body(h_ref, mi0_ref, mi1_ref, vi_ref, vu0_ref, vu1_ref, x_ref,
             vfw1_ref, vfb1_ref, vfw2_ref, vfb2_ref,
             nfw1a_ref, nfw1b_ref, nfb1_ref, nfw2_ref, nfb2_ref,
             h_out, x_out, v_out):
    hb = h_ref[...]
    mi = mi0_ref[...] + mi1_ref[...]
    vu = vu0_ref[...] + vu1_ref[...]
    a = _silu(jnp.dot(hb, vfw1_ref[...], preferred_element_type=_f32)
              + vfb1_ref[...])
    vpart = jnp.dot(a, vfw2_ref[...], preferred_element_type=_f32) + vfb2_ref[...]
    v_new = vpart * vi_ref[...] + vu
    nb = _silu(jnp.dot(hb, nfw1a_ref[...], preferred_element_type=_f32)
               + jnp.dot(mi, nfw1b_ref[...], preferred_element_type=_f32)
               + nfb1_ref[...])
    h_out[...] = jnp.dot(nb, nfw2_ref[...], preferred_element_type=_f32) \
        + nfb2_ref[...] + hb
    x_out[...] = x_ref[...] + v_new
    v_out[...] = v_new


def _t2(h, mi0, mi1, vi, vu0, vu1, x,
        vfw1, vfb1, vfw2, vfb2, nfw1a, nfw1b, nfb1, nfw2, nfb2, bn=2000):
    grid = (N // bn,)
    full = lambda shape: pl.BlockSpec(shape, lambda i: tuple(0 for _ in shape))
    row = lambda w: pl.BlockSpec((bn, w), lambda i: (i, 0))
    return pl.pallas_call(
        _t2_body,
        grid=grid,
        in_specs=[
            row(F), row(H), row(H), row(V), row(V), row(V), row(V),
            full((F, H)), full((1, H)), full((H, V)), full((1, V)),
            full((F, H)), full((F, H)), full((1, H)), full((H, F)),
            full((1, F)),
        ],
        out_specs=[row(F), row(V), row(V)],
        out_shape=[
            jax.ShapeDtypeStruct((N, F), _f32),
            jax.ShapeDtypeStruct((N, V), _f32),
            jax.ShapeDtypeStruct((N, V), _f32),
        ],
    )(h, mi0, mi1, vi, vu0, vu1, x,
      vfw1, vfb1, vfw2, vfb2, nfw1a, nfw1b, nfb1, nfw2, nfb2)


def kernel(h, x, v_init, edge_index, edge_attr,
           ef_w1, ef_b1, ef_w2, ef_b2,
           cf_w1, cf_b1, cf_w2, cf_b2,
           vf_w1, vf_b1, vf_w2, vf_b2,
           nf_w1, nf_b1, nf_w2, nf_b2, coord_scaling):
    h = h.astype(_f32)
    x = x.astype(_f32)
    src = edge_index[0].astype(jnp.int32)
    dst = edge_index[1].astype(jnp.int32)

    w1a = ef_w1[:F]
    w1b = ef_w1[F:2 * F]
    w1c = ef_w1[2 * F:2 * F + 1]
    w1d = ef_w1[2 * F + 1:]

    a, b = _t0(h, w1a, w1b)

    gs, gd, xs, xd = _sc_gather(a, b, x, src, dst)

    r1 = lambda v: v.reshape(1, -1)
    cs = coord_scaling.reshape(1, 1).astype(_f32)
    m_ij, cu = _t1(gs, gd, edge_attr, xs, xd,
                   w1d, w1c, r1(ef_b1), ef_w2, r1(ef_b2),
                   cf_w1, cf_b1.reshape(1, 1), cf_w2, cf_b2.reshape(1, 1), cs)

    zeros_h = jnp.zeros((N, H), _f32)
    zeros_v = jnp.zeros((N, V), _f32)
    mi_parts, vu_parts = _sc_scatter(m_ij, cu, dst, zeros_h, zeros_v)

    nfw1a = nf_w1[:F]
    nfw1b = nf_w1[F:]
    h_new, x_new, v_new = _t2(
        h, mi_parts[0], mi_parts[1], v_init, vu_parts[0], vu_parts[1], x,
        vf_w1, r1(vf_b1), vf_w2, r1(vf_b2),
        nfw1a, nfw1b, r1(nf_b1), nf_w2, r1(nf_b2))
    return (h_new, x_new, v_new)


# speedup vs baseline: 2.4616x; 2.4616x over previous
import functools

import jax
import jax.numpy as jnp
from jax import lax
from jax.experimental import pallas as pl
from jax.experimental.pallas import tpu as pltpu
from jax.experimental.pallas import tpu_sc as plsc

N, E, F, H, EA, V = 10000, 320000, 128, 128, 16, 3

NC, NS = 2, 16
NW = NC * NS
E_LOCAL = E // NW
CH = 80
NCHUNK = E_LOCAL // CH
SLAB = 1000
NSLABS = N // SLAB
LANES = 16
NVP = 30720
W = NVP // NS

_f32 = jnp.float32


def _silu(z):
    return z * jax.nn.sigmoid(z)


def _t0_body(h_ref, w1a_ref, w1b_ref, a_ref, b_ref):
    hb = h_ref[...]
    a_ref[...] = jnp.dot(hb, w1a_ref[...], preferred_element_type=_f32)
    b_ref[...] = jnp.dot(hb, w1b_ref[...], preferred_element_type=_f32)


def _t0(h, w1a, w1b, bn=2000):
    grid = (N // bn,)
    return pl.pallas_call(
        _t0_body,
        grid=grid,
        in_specs=[
            pl.BlockSpec((bn, F), lambda i: (i, 0)),
            pl.BlockSpec((F, H), lambda i: (0, 0)),
            pl.BlockSpec((F, H), lambda i: (0, 0)),
        ],
        out_specs=[
            pl.BlockSpec((bn, H), lambda i: (i, 0)),
            pl.BlockSpec((bn, H), lambda i: (i, 0)),
        ],
        out_shape=[
            jax.ShapeDtypeStruct((N, H), _f32),
            jax.ShapeDtypeStruct((N, H), _f32),
        ],
    )(h, w1a, w1b)


def _sc_gather(a, b, x, src, dst):
    mesh = plsc.VectorSubcoreMesh(core_axis_name="c", subcore_axis_name="s")

    @functools.partial(
        pl.kernel,
        mesh=mesh,
        compiler_params=pltpu.CompilerParams(needs_layout_passes=False),
        out_type=(
            jax.ShapeDtypeStruct((E, H), _f32),
            jax.ShapeDtypeStruct((E, H), _f32),
            jax.ShapeDtypeStruct((E,), _f32),
            jax.ShapeDtypeStruct((E,), _f32),
            jax.ShapeDtypeStruct((E,), _f32),
        ),
        scratch_types=[
            pltpu.VMEM((N * V,), _f32),
            pltpu.VMEM((CH,), jnp.int32),
            pltpu.VMEM((CH,), jnp.int32),
            pltpu.VMEM((CH, H), _f32),
            pltpu.VMEM((CH, H), _f32),
            pltpu.VMEM((CH,), _f32),
            pltpu.VMEM((CH,), _f32),
            pltpu.VMEM((CH,), _f32),
            pltpu.SemaphoreType.DMA,
            pltpu.SemaphoreType.DMA,
        ],
    )
    def kern(a_hbm, b_hbm, x_hbm, src_hbm, dst_hbm,
             gs_out, gd_out, dx_out, dy_out, dz_out,
             x_v, src_v, dst_v, buf_a, buf_b, bdx, bdy, bdz, sem_g, sem_w):
        wid = lax.axis_index("s") * NC + lax.axis_index("c")
        base = wid * E_LOCAL
        pltpu.sync_copy(x_hbm, x_v)

        def chunk(i, carry):
            off = base + i * CH
            pltpu.sync_copy(src_hbm.at[pl.ds(off, CH)], src_v)
            pltpu.sync_copy(dst_hbm.at[pl.ds(off, CH)], dst_v)
            g1 = pltpu.async_copy(a_hbm.at[src_v], buf_a, sem_g)
            g2 = pltpu.async_copy(b_hbm.at[dst_v], buf_b, sem_g)
            for j in range(CH // LANES):
                sv = src_v[pl.ds(j * LANES, LANES)] * V
                dv = dst_v[pl.ds(j * LANES, LANES)] * V
                for c, bufc in ((0, bdx), (1, bdy), (2, bdz)):
                    xs = plsc.load_gather(x_v, [sv + c])
                    xd = plsc.load_gather(x_v, [dv + c])
                    bufc[pl.ds(j * LANES, LANES)] = xd - xs
            g1.wait()
            g2.wait()
            w1 = pltpu.async_copy(buf_a, gs_out.at[pl.ds(off, CH)], sem_w)
            w2 = pltpu.async_copy(buf_b, gd_out.at[pl.ds(off, CH)], sem_w)
            w3 = pltpu.async_copy(bdx, dx_out.at[pl.ds(off, CH)], sem_w)
            w4 = pltpu.async_copy(bdy, dy_out.at[pl.ds(off, CH)], sem_w)
            w5 = pltpu.async_copy(bdz, dz_out.at[pl.ds(off, CH)], sem_w)
            w1.wait()
            w2.wait()
            w3.wait()
            w4.wait()
            w5.wait()
            return carry

        lax.fori_loop(0, NCHUNK, chunk, 0)

    return kern(a, b, x, src, dst)


def _t1_body(gs_ref, gd_ref, ea_ref, dx_ref, dy_ref, dz_ref,
             w1d_ref, w1c_ref, b1_ref, w2_ref, b2_ref,
             cfw1_ref, cfb1_ref, cfw2_ref, cfb2_ref, cs_ref,
             m_ref, cux_ref, cuy_ref, cuz_ref):
    dx = dx_ref[...]
    dy = dy_ref[...]
    dz = dz_ref[...]
    xsq = dx * dx + dy * dy + dz * dz
    z1 = (gs_ref[...] + gd_ref[...]
          + jnp.dot(ea_ref[...], w1d_ref[...], preferred_element_type=_f32)
          + xsq * w1c_ref[...]
          + b1_ref[...])
    m1 = _silu(z1)
    t2 = jnp.dot(m1, w2_ref[...], preferred_element_type=_f32) + b2_ref[...]
    m_ij = _silu(t2)
    g1 = jnp.dot(m_ij, cfw1_ref[...], preferred_element_type=_f32) + cfb1_ref[...]
    gate = cs_ref[0, 0] * (_silu(g1) * cfw2_ref[0, 0] + cfb2_ref[0, 0])
    m_ref[...] = m_ij
    cux_ref[...] = gate * dx
    cuy_ref[...] = gate * dy
    cuz_ref[...] = gate * dz


def _t1(gs, gd, ea, dx, dy, dz, w1d, w1c, b1, w2, b2,
        cfw1, cfb1, cfw2, cfb2, cs, be=2000):
    grid = (E // be,)
    full = lambda shape: pl.BlockSpec(shape, lambda i: tuple(0 for _ in shape))
    col = pl.BlockSpec((be, 1), lambda i: (i, 0))
    return pl.pallas_call(
        _t1_body,
        grid=grid,
        in_specs=[
            pl.BlockSpec((be, H), lambda i: (i, 0)),
            pl.BlockSpec((be, H), lambda i: (i, 0)),
            pl.BlockSpec((be, EA), lambda i: (i, 0)),
            col, col, col,
            full((EA, H)),
            full((1, H)),
            full((1, H)),
            full((H, F)),
            full((1, F)),
            full((F, 1)),
            full((1, 1)),
            full((1, 1)),
            full((1, 1)),
            full((1, 1)),
        ],
        out_specs=[pl.BlockSpec((be, F), lambda i: (i, 0)), col, col, col],
        out_shape=[
            jax.ShapeDtypeStruct((E, F), _f32),
            jax.ShapeDtypeStruct((E, 1), _f32),
            jax.ShapeDtypeStruct((E, 1), _f32),
            jax.ShapeDtypeStruct((E, 1), _f32),
        ],
    )(gs, gd, ea, dx, dy, dz, w1d, w1c, b1, w2, b2,
      cfw1, cfb1, cfw2, cfb2, cs)


def _sc_scatter(m_ij, cux, cuy, cuz, dst, zeros_h):
    mesh = plsc.VectorSubcoreMesh(core_axis_name="c", subcore_axis_name="s")
    nzero = NVP // LANES

    @functools.partial(
        pl.kernel,
        mesh=mesh,
        compiler_params=pltpu.CompilerParams(needs_layout_passes=False),
        out_type=(
            jax.ShapeDtypeStruct((NC, N, H), _f32),
            jax.ShapeDtypeStruct((NW, NVP), _f32),
        ),
        scratch_types=[
            pltpu.VMEM((CH,), jnp.int32),
            pltpu.VMEM((CH, H), _f32),
            pltpu.VMEM((CH,), _f32),
            pltpu.VMEM((CH,), _f32),
            pltpu.VMEM((CH,), _f32),
            pltpu.VMEM((NVP,), _f32),
            pltpu.VMEM_SHARED((N, H), _f32),
        ],
    )
    def kern(m_hbm, cux_hbm, cuy_hbm, cuz_hbm, dst_hbm, zh_hbm,
             mi_out, vu_out,
             idx_v, buf_m, bcx, bcy, bcz, acc_vu, acc_mi):
        cid = lax.axis_index("c")
        sid = lax.axis_index("s")
        wid = sid * NC + cid
        r0 = sid * SLAB
        @pl.when(sid < NSLABS)
        def _zero_slab():
            pltpu.sync_copy(zh_hbm.at[pl.ds(r0, SLAB)],
                            acc_mi.at[pl.ds(r0, SLAB)])

        zv = jnp.zeros((LANES,), _f32)

        def zero_body(k, carry):
            acc_vu[pl.ds(k * LANES, LANES)] = zv
            return carry

        lax.fori_loop(0, nzero, zero_body, 0)
        plsc.subcore_barrier()

        base = wid * E_LOCAL

        def chunk(i, carry):
            off = base + i * CH
            pltpu.sync_copy(dst_hbm.at[pl.ds(off, CH)], idx_v)
            pltpu.sync_copy(m_hbm.at[pl.ds(off, CH)], buf_m)
            pltpu.sync_copy(cux_hbm.at[pl.ds(off, CH)], bcx)
            pltpu.sync_copy(cuy_hbm.at[pl.ds(off, CH)], bcy)
            pltpu.sync_copy(cuz_hbm.at[pl.ds(off, CH)], bcz)
            pltpu.sync_copy(buf_m, acc_mi.at[idx_v], add=True)
            for j in range(CH // LANES):
                dv = idx_v[pl.ds(j * LANES, LANES)]
                fid = dv * V
                for c, bufc in ((0, bcx), (1, bcy), (2, bcz)):
                    val = bufc[pl.ds(j * LANES, LANES)]
                    plsc.addupdate_scatter(acc_vu, [fid + c], val)
            return carry

        lax.fori_loop(0, NCHUNK, chunk, 0)
        plsc.subcore_barrier()

        @pl.when(sid < NSLABS)
        def _write_slab():
            pltpu.sync_copy(acc_mi.at[pl.ds(r0, SLAB)],
                            mi_out.at[cid, pl.ds(r0, SLAB)])

        pltpu.sync_copy(acc_vu, vu_out.at[wid])

    return kern(m_ij, cux, cuy, cuz, dst, zeros_h)


def _t1b_body(vu_ref, out_ref):
    out_ref[...] = jnp.sum(vu_ref[...], axis=0, keepdims=True)


def _t1b(vu_parts, bnv=3840):
    grid = (NVP // bnv,)
    return pl.pallas_call(
        _t1b_body,
        grid=grid,
        in_specs=[pl.BlockSpec((NW, bnv), lambda i: (0, i))],
        out_specs=pl.BlockSpec((1, bnv), lambda i: (0, i)),
        out_shape=jax.ShapeDtypeStruct((1, NVP), _f32),
    )(vu_parts)


def _t2_body(h_ref, mi0_ref, mi1_ref, vi_ref, vu_ref, x_ref,
             vfw1_ref, vfb1_ref, vfw2_ref, vfb2_ref,
             nfw1a_ref, nfw1b_ref, nfb1_ref, nfw2_ref, nfb2_ref,
             h_out, x_out, v_out):
    hb = h_ref[...]
    mi = mi0_ref[...] + mi1_ref[...]
    vu = vu_ref[...]
    a = _silu(jnp.dot(hb, vfw1_ref[...], preferred_element_type=_f32)
              + vfb1_ref[...])
    vpart = jnp.dot(a, vfw2_ref[...], preferred_element_type=_f32) + vfb2_ref[...]
    v_new = vpart * vi_ref[...] + vu
    nb = _silu(jnp.dot(hb, nfw1a_ref[...], preferred_element_type=_f32)
               + jnp.dot(mi, nfw1b_ref[...], preferred_element_type=_f32)
               + nfb1_ref[...])
    h_out[...] = jnp.dot(nb, nfw2_ref[...], preferred_element_type=_f32) \
        + nfb2_ref[...] + hb
    x_out[...] = x_ref[...] + v_new
    v_out[...] = v_new


def _t2(h, mi0, mi1, vi, vu, x,
        vfw1, vfb1, vfw2, vfb2, nfw1a, nfw1b, nfb1, nfw2, nfb2, bn=2000):
    grid = (N // bn,)
    full = lambda shape: pl.BlockSpec(shape, lambda i: tuple(0 for _ in shape))
    row = lambda w: pl.BlockSpec((bn, w), lambda i: (i, 0))
    return pl.pallas_call(
        _t2_body,
        grid=grid,
        in_specs=[
            row(F), row(H), row(H), row(V), row(V), row(V),
            full((F, H)), full((1, H)), full((H, V)), full((1, V)),
            full((F, H)), full((F, H)), full((1, H)), full((H, F)),
            full((1, F)),
        ],
        out_specs=[row(F), row(V), row(V)],
        out_shape=[
            jax.ShapeDtypeStruct((N, F), _f32),
            jax.ShapeDtypeStruct((N, V), _f32),
            jax.ShapeDtypeStruct((N, V), _f32),
        ],
    )(h, mi0, mi1, vi, vu, x,
      vfw1, vfb1, vfw2, vfb2, nfw1a, nfw1b, nfb1, nfw2, nfb2)


def kernel(h, x, v_init, edge_index, edge_attr,
           ef_w1, ef_b1, ef_w2, ef_b2,
           cf_w1, cf_b1, cf_w2, cf_b2,
           vf_w1, vf_b1, vf_w2, vf_b2,
           nf_w1, nf_b1, nf_w2, nf_b2, coord_scaling):
    h = h.astype(_f32)
    x = x.astype(_f32)
    src = edge_index[0].astype(jnp.int32)
    dst = edge_index[1].astype(jnp.int32)

    w1a = ef_w1[:F]
    w1b = ef_w1[F:2 * F]
    w1c = ef_w1[2 * F:2 * F + 1]
    w1d = ef_w1[2 * F + 1:]

    a, b = _t0(h, w1a, w1b)

    gs, gd, dxv, dyv, dzv = _sc_gather(a, b, x.reshape(N * V), src, dst)

    r1 = lambda v: v.reshape(1, -1)
    c1 = lambda v: v.reshape(-1, 1)
    cs = coord_scaling.reshape(1, 1).astype(_f32)
    m_ij, cux, cuy, cuz = _t1(
        gs, gd, edge_attr, c1(dxv), c1(dyv), c1(dzv),
        w1d, w1c, r1(ef_b1), ef_w2, r1(ef_b2),
        cf_w1, cf_b1.reshape(1, 1), cf_w2, cf_b2.reshape(1, 1), cs)

    zeros_h = jnp.zeros((N, H), _f32)
    mi_parts, vu_flat = _sc_scatter(
        m_ij, cux.reshape(E), cuy.reshape(E), cuz.reshape(E), dst, zeros_h)

    vu = _t1b(vu_flat)[0, :N * V].reshape(N, V)
    nfw1a = nf_w1[:F]
    nfw1b = nf_w1[F:]
    h_new, x_new, v_new = _t2(
        h, mi_parts[0], mi_parts[1], v_init, vu, x,
        vf_w1, r1(vf_b1), vf_w2, r1(vf_b2),
        nfw1a, nfw1b, r1(nf_b1), nf_w2, r1(nf_b2))
    return (h_new, x_new, v_new)
